# baseline (speedup 1.0000x reference)
import functools

import jax
import jax.numpy as jnp
from jax import lax
from jax.experimental import pallas as pl
from jax.experimental.pallas import tpu as pltpu
from jax.experimental.pallas import tpu_sc as plsc

B = 4096
L = 200
D = 64

NC = 2
NS = 16
NW = NC * NS

ROWS = B * L
RPW = ROWS // NW
C = 256
NCH = RPW // C
G = C // 128

VOCAB = 1000000
CR = 256
NCHP = VOCAB // CR
PTAIL0 = NCHP * CR
PTAILR = VOCAB - PTAIL0
CPW = ((NCHP + NW - 1) // NW + 1) // 2 * 2


def _make_prekernel():
    mesh = plsc.VectorSubcoreMesh(core_axis_name="c", subcore_axis_name="s")

    @functools.partial(
        pl.kernel,
        mesh=mesh,
        compiler_params=pltpu.CompilerParams(use_tc_tiling_on_sc=True),
        out_type=jax.ShapeDtypeStruct((VOCAB // 2, 128), jnp.float32),
        scratch_types=[
            pltpu.VMEM((CR, D), jnp.float32),
            pltpu.VMEM((CR, D), jnp.float32),
            pltpu.VMEM((CR // 2, 128), jnp.float32),
            pltpu.VMEM((CR // 2, 128), jnp.float32),
            pltpu.VMEM((PTAILR, D), jnp.float32),
            pltpu.VMEM((PTAILR // 2, 128), jnp.float32),
            pltpu.SemaphoreType.DMA,
            pltpu.SemaphoreType.DMA,
            pltpu.SemaphoreType.DMA,
            pltpu.SemaphoreType.DMA,
        ],
    )
    def pre_kernel(tokp_hbm, out_hbm, b0, b1, t0, t1, tb, tp,
                   isem0, isem1, osem0, osem1):
        wid = lax.axis_index("s") * NC + lax.axis_index("c")
        blk = (b0, b1)
        tr = (t0, t1)
        isem = (isem0, isem1)
        osem = (osem0, osem1)

        def chunk_idx(t):
            w = wid + t * NW
            return lax.select(w >= NCHP, NCHP - 1, w)

        def start_in(t, p):
            row = pl.multiple_of(chunk_idx(t) * CR, 8)
            pltpu.async_copy(tokp_hbm.at[pl.ds(row, CR)], blk[p], isem[p])

        def wait_in(p):
            pltpu.make_async_copy(
                tokp_hbm.at[pl.ds(0, CR)], blk[p], isem[p]
            ).wait()

        def wait_out(p):
            pltpu.make_async_copy(
                tr[p], out_hbm.at[pl.ds(0, CR // 2)], osem[p]
            ).wait()

        def repack(src, dst, npairs):
            @plsc.parallel_loop(0, npairs, 1, unroll=4)
            def _pair(j):
                for h in range(D // 16):
                    dst[j, pl.ds(16 * h, 16)] = src[2 * j, pl.ds(16 * h, 16)]
                    dst[j, pl.ds(D + 16 * h, 16)] = src[
                        2 * j + 1, pl.ds(16 * h, 16)
                    ]

        start_in(0, 0)

        def pair_body(g2, carry):
            for p in range(2):
                t = 2 * g2 + p
                pl.when(t + 1 < CPW)(lambda: start_in(t + 1, 1 - p))
                wait_in(p)
                pl.when(t >= 2)(lambda: wait_out(p))
                repack(blk[p], tr[p], CR // 2)
                prow = pl.multiple_of(chunk_idx(t) * (CR // 2), 8)
                pltpu.async_copy(
                    tr[p], out_hbm.at[pl.ds(prow, CR // 2)], osem[p]
                )
            return carry

        lax.fori_loop(0, CPW // 2, pair_body, 0)
        wait_out(0)
        wait_out(1)

        pltpu.sync_copy(tokp_hbm.at[pl.ds(PTAIL0, PTAILR)], tb)
        repack(tb, tp, PTAILR // 2)
        pltpu.sync_copy(tp, out_hbm.at[pl.ds(PTAIL0 // 2, PTAILR // 2)])

    return pre_kernel


def _make_kernel():
    mesh = plsc.VectorSubcoreMesh(core_axis_name="c", subcore_axis_name="s")

    @functools.partial(
        pl.kernel,
        mesh=mesh,
        compiler_params=pltpu.CompilerParams(use_tc_tiling_on_sc=False),
        out_type=jax.ShapeDtypeStruct((ROWS, 2 * D), jnp.float32),
        scratch_types=[
            pltpu.VMEM((RPW,), jnp.int32),
            pltpu.VMEM((L * D,), jnp.float32),
            pltpu.VMEM((C, D), jnp.float32),
            pltpu.VMEM((C, D), jnp.float32),
            pltpu.VMEM((C, D), jnp.float32),
            pltpu.VMEM((C, D), jnp.float32),
            pltpu.SemaphoreType.DMA,
            pltpu.SemaphoreType.DMA,
            pltpu.SemaphoreType.DMA,
            pltpu.SemaphoreType.DMA,
        ],
    )
    def emb_kernel(xf_hbm, tok_hbm, posf_hbm, out_hbm,
                   idx_v, pos_v, g0, g1, o0, o1,
                   gsem0, gsem1, osem0, osem1):
        wid = lax.axis_index("s") * NC + lax.axis_index("c")
        base = wid * RPW
        g = (g0, g1)
        o = (o0, o1)
        gsem = (gsem0, gsem1)
        osem = (osem0, osem1)

        pltpu.sync_copy(xf_hbm.at[pl.ds(base, RPW)], idx_v)
        pltpu.sync_copy(posf_hbm, pos_v)

        def start_gathers(i, b):
            for k in range(G):
                ioff = pl.multiple_of(i * C + k * 128, 8)
                pltpu.async_copy(
                    tok_hbm.at[idx_v.at[pl.ds(ioff, 128)]],
                    g[b].at[pl.ds(k * 128, 128)],
                    gsem[b],
                )

        def wait_gathers(b):
            pltpu.make_async_copy(tok_hbm.at[pl.ds(0, C)], g[b], gsem[b]).wait()

        def wait_store(b):
            pltpu.make_async_copy(
                o[b], out_hbm.at[pl.ds(0, C), pl.ds(0, D)], osem[b]
            ).wait()

        def add_pos(i, b):
            p0 = lax.rem(i * C, L)

            @plsc.parallel_loop(0, C, 1, unroll=8)
            def _row_body(r):
                p = p0 + r
                p = lax.select(p >= L, p - L, p)
                p = lax.select(p >= L, p - L, p)
                pb = pl.multiple_of(p * D, 8)
                for k in range(D // 16):
                    o[b][r, pl.ds(k * 16, 16)] = (
                        g[b][r, pl.ds(k * 16, 16)]
                        + pos_v[pl.ds(pb + k * 16, 16)]
                    )

        start_gathers(0, 0)

        def pair_body(h, carry):
            for b in range(2):
                i = 2 * h + b
                pl.when(i + 1 < NCH)(lambda: start_gathers(i + 1, 1 - b))
                wait_gathers(b)
                pl.when(i >= 2)(lambda: wait_store(b))
                add_pos(i, b)
                pltpu.async_copy(
                    o[b],
                    out_hbm.at[pl.ds(base + i * C, C), pl.ds(0, D)],
                    osem[b],
                )
            return carry

        lax.fori_loop(0, NCH // 2, pair_body, 0)
        wait_store(0)
        wait_store(1)

    return emb_kernel


_emb = _make_kernel()
_pre = _make_prekernel()


@jax.jit
def kernel(x, token_table, pos_table):
    xf = x.reshape(-1)
    posf = pos_table[:L].reshape(-1)
    tok_lin = _pre(token_table).reshape(VOCAB, D)
    out2 = _emb(xf, tok_lin, posf)
    return out2[:, :D].reshape(B, L, D)


# speedup vs baseline: 1.0129x; 1.0129x over previous
import functools

import jax
import jax.numpy as jnp
from jax import lax
from jax.experimental import pallas as pl
from jax.experimental.pallas import tpu as pltpu
from jax.experimental.pallas import tpu_sc as plsc

B = 4096
L = 200
D = 64

NC = 2
NS = 16
NW = NC * NS

ROWS = B * L
RPW = ROWS // NW
C = 256
NCH = RPW // C
G = C // 128


def _make_kernel():
    mesh = plsc.VectorSubcoreMesh(core_axis_name="c", subcore_axis_name="s")

    @functools.partial(
        pl.kernel,
        mesh=mesh,
        compiler_params=pltpu.CompilerParams(use_tc_tiling_on_sc=False),
        out_type=jax.ShapeDtypeStruct((ROWS, 2 * D), jnp.float32),
        scratch_types=[
            pltpu.VMEM((RPW,), jnp.int32),
            pltpu.VMEM((L * D,), jnp.float32),
            pltpu.VMEM((C, D), jnp.float32),
            pltpu.VMEM((C, D), jnp.float32),
            pltpu.VMEM((C, D), jnp.float32),
            pltpu.VMEM((C, D), jnp.float32),
            pltpu.SemaphoreType.DMA,
            pltpu.SemaphoreType.DMA,
            pltpu.SemaphoreType.DMA,
            pltpu.SemaphoreType.DMA,
        ],
    )
    def emb_kernel(xf_hbm, tok_hbm, posf_hbm, out_hbm,
                   idx_v, pos_v, g0, g1, o0, o1,
                   gsem0, gsem1, osem0, osem1):
        wid = lax.axis_index("s") * NC + lax.axis_index("c")
        base = wid * RPW
        g = (g0, g1)
        o = (o0, o1)
        gsem = (gsem0, gsem1)
        osem = (osem0, osem1)

        pltpu.sync_copy(xf_hbm.at[pl.ds(base, RPW)], idx_v)
        pltpu.sync_copy(posf_hbm, pos_v)

        def start_gathers(i, b):
            for k in range(G):
                ioff = pl.multiple_of(i * C + k * 128, 8)
                pltpu.async_copy(
                    tok_hbm.at[idx_v.at[pl.ds(ioff, 128)]],
                    g[b].at[pl.ds(k * 128, 128)],
                    gsem[b],
                )

        def wait_gathers(b):
            pltpu.make_async_copy(tok_hbm.at[pl.ds(0, C)], g[b], gsem[b]).wait()

        def wait_store(b):
            pltpu.make_async_copy(
                o[b], out_hbm.at[pl.ds(0, C), pl.ds(0, D)], osem[b]
            ).wait()

        def add_pos(i, b):
            p0 = lax.rem(i * C, L)

            @plsc.parallel_loop(0, C, 1, unroll=8)
            def _row_body(r):
                p = p0 + r
                p = lax.select(p >= L, p - L, p)
                p = lax.select(p >= L, p - L, p)
                pb = pl.multiple_of(p * D, 8)
                for k in range(D // 16):
                    o[b][r, pl.ds(k * 16, 16)] = (
                        g[b][r, pl.ds(k * 16, 16)]
                        + pos_v[pl.ds(pb + k * 16, 16)]
                    )

        start_gathers(0, 0)

        def pair_body(h, carry):
            for b in range(2):
                i = 2 * h + b
                pl.when(i + 1 < NCH)(lambda: start_gathers(i + 1, 1 - b))
                wait_gathers(b)
                pl.when(i >= 2)(lambda: wait_store(b))
                add_pos(i, b)
                pltpu.async_copy(
                    o[b],
                    out_hbm.at[pl.ds(base + i * C, C), pl.ds(0, D)],
                    osem[b],
                )
            return carry

        lax.fori_loop(0, NCH // 2, pair_body, 0)
        wait_store(0)
        wait_store(1)

    return emb_kernel


_emb = _make_kernel()


@jax.jit
def kernel(x, token_table, pos_table):
    xf = x.reshape(-1)
    posf = pos_table[:L].reshape(-1)
    out2 = _emb(xf, token_table, posf)
    return out2[:, :D].reshape(B, L, D)
